# baseline (speedup 1.0000x reference)
import math

import jax
import jax.numpy as jnp
import numpy as np
from jax import lax
from jax.experimental import pallas as pl
from jax.experimental.pallas import tpu as pltpu
from jax.experimental.pallas import tpu_sc as plsc

VOCAB = 100000
EMBED = 64
OUT_DIM = 128
B = 4096
L = 200
MAX_LEN = 200

NUM_CORES = 2
NUM_SUBCORES = 16
NW = NUM_CORES * NUM_SUBCORES
B_PER_W = B // NW
G0, G1 = 128, 72
LPAD = 256
CHUNK = B_PER_W * LPAD


def _positional_table():
    position = np.arange(MAX_LEN, dtype=np.float32)[:, None]
    div_term = np.exp(
        np.arange(0, EMBED, 2, dtype=np.float32) * (-math.log(10000.0) / EMBED))
    pe = np.zeros((MAX_LEN, EMBED), dtype=np.float32)
    pe[:, 0::2] = np.sin(position * div_term)
    pe[:, 1::2] = np.cos(position * div_term)
    return pe


_PE = _positional_table()


def _sc_gather_sum(emb_table, idx_pad_flat):
    mesh = plsc.VectorSubcoreMesh(core_axis_name="c", subcore_axis_name="s")

    def body(table_hbm, idx_hbm, out_hbm, idx_v, rows_v, out_v, sem0, sem1):
        wid = lax.axis_index("s") * NUM_CORES + lax.axis_index("c")
        pltpu.sync_copy(idx_hbm.at[pl.ds(wid * CHUNK, CHUNK)], idx_v)

        def gather(slot, r, sem):
            base = r * LPAD
            pltpu.make_async_copy(
                table_hbm.at[idx_v.at[pl.ds(base, G0)]],
                rows_v.at[slot, pl.ds(0, G0)], sem).start()
            pltpu.make_async_copy(
                table_hbm.at[idx_v.at[pl.ds(base + G0, G1)]],
                rows_v.at[slot, pl.ds(G0, G1)], sem).start()

        def drain(slot, sem):
            pltpu.make_async_copy(
                table_hbm.at[idx_v.at[pl.ds(0, G0)]],
                rows_v.at[slot, pl.ds(0, G0)], sem).wait()
            pltpu.make_async_copy(
                table_hbm.at[idx_v.at[pl.ds(0, G1)]],
                rows_v.at[slot, pl.ds(G0, G1)], sem).wait()

        def reduce_into(slot, r):
            def red(l, accs):
                b0, b1, b2, b3, c0, c1, c2, c3 = accs
                row = 4 * l
                b0 = b0 + rows_v[slot, row, pl.ds(0, 16)]
                b1 = b1 + rows_v[slot, row, pl.ds(16, 16)]
                b2 = b2 + rows_v[slot, row, pl.ds(32, 16)]
                b3 = b3 + rows_v[slot, row, pl.ds(48, 16)]
                c0 = c0 + rows_v[slot, row + 1, pl.ds(0, 16)]
                c1 = c1 + rows_v[slot, row + 1, pl.ds(16, 16)]
                c2 = c2 + rows_v[slot, row + 1, pl.ds(32, 16)]
                c3 = c3 + rows_v[slot, row + 1, pl.ds(48, 16)]
                b0 = b0 + rows_v[slot, row + 2, pl.ds(0, 16)]
                b1 = b1 + rows_v[slot, row + 2, pl.ds(16, 16)]
                b2 = b2 + rows_v[slot, row + 2, pl.ds(32, 16)]
                b3 = b3 + rows_v[slot, row + 2, pl.ds(48, 16)]
                c0 = c0 + rows_v[slot, row + 3, pl.ds(0, 16)]
                c1 = c1 + rows_v[slot, row + 3, pl.ds(16, 16)]
                c2 = c2 + rows_v[slot, row + 3, pl.ds(32, 16)]
                c3 = c3 + rows_v[slot, row + 3, pl.ds(48, 16)]
                return (b0, b1, b2, b3, c0, c1, c2, c3)

            z = jnp.zeros((16,), jnp.float32)
            accs = lax.fori_loop(0, L // 4, red, (z,) * 8)
            out_v[r, pl.ds(0, 16)] = accs[0] + accs[4]
            out_v[r, pl.ds(16, 16)] = accs[1] + accs[5]
            out_v[r, pl.ds(32, 16)] = accs[2] + accs[6]
            out_v[r, pl.ds(48, 16)] = accs[3] + accs[7]

        gather(0, 0, sem0)

        def pair_body(i, carry):
            r0 = 2 * i
            gather(1, r0 + 1, sem1)
            drain(0, sem0)
            reduce_into(0, r0)

            @pl.when(r0 + 2 < B_PER_W)
            def _():
                gather(0, r0 + 2, sem0)

            drain(1, sem1)
            reduce_into(1, r0 + 1)
            return carry

        lax.fori_loop(0, B_PER_W // 2, pair_body, 0)
        pltpu.sync_copy(out_v, out_hbm.at[pl.ds(wid * B_PER_W, B_PER_W)])

    call = pl.kernel(
        body,
        out_type=jax.ShapeDtypeStruct((B, EMBED), jnp.float32),
        mesh=mesh,
        compiler_params=pltpu.CompilerParams(use_tc_tiling_on_sc=False),
        scratch_types=[
            pltpu.VMEM((CHUNK,), jnp.int32),
            pltpu.VMEM((2, L, EMBED), jnp.float32),
            pltpu.VMEM((B_PER_W, EMBED), jnp.float32),
            pltpu.SemaphoreType.DMA,
            pltpu.SemaphoreType.DMA,
        ],
    )
    return call(emb_table, idx_pad_flat)


TROWS = 1024
TGRID = -(-VOCAB // TROWS)


VOCAB_PAD = TGRID * TROWS


def _table_lin_body(tt_ref, out_ref):
    t = tt_ref[...]
    a = t[:, : TROWS // 2].T
    b = t[:, TROWS // 2 :].T
    out_ref[...] = jnp.concatenate([a, b], axis=1)


def _table_linear(emb_table):
    tt = emb_table.T
    out = pl.pallas_call(
        _table_lin_body,
        grid=(TGRID,),
        in_specs=[pl.BlockSpec((EMBED, TROWS), lambda i: (0, i))],
        out_specs=pl.BlockSpec((TROWS // 2, 2 * EMBED), lambda i: (i, 0)),
        out_shape=jax.ShapeDtypeStruct((VOCAB_PAD // 2, 2 * EMBED), jnp.float32),
    )(tt)
    return out.reshape(VOCAB_PAD, EMBED)


def _tc_combine_body(idx_ref, sums_ref, pe_ref, wt_ref, b_ref, e1_ref, out_ref):
    idx = idx_ref[...]
    m = jnp.logical_and(idx != 0, idx != 1).astype(jnp.float32)
    n_eos = jnp.sum((idx == 1).astype(jnp.float32), axis=1, keepdims=True)
    count = jnp.sum(m, axis=1, keepdims=True)
    pe_sum = jnp.dot(m, pe_ref[...], preferred_element_type=jnp.float32)
    s_masked = sums_ref[...] - n_eos * e1_ref[...]
    pooled = (8.0 * s_masked + pe_sum) / (count + 1e-6)
    out_ref[...] = (
        jnp.dot(pooled, wt_ref[...], preferred_element_type=jnp.float32)
        + b_ref[...])


def _tc_combine(idx, sums, wt, bvec, e1):
    return pl.pallas_call(
        _tc_combine_body,
        out_shape=jax.ShapeDtypeStruct((B, OUT_DIM), jnp.float32),
    )(idx, sums, jnp.asarray(_PE), wt, bvec, e1)


def kernel(src_tok_idxs, emb_table, W, b):
    idx = src_tok_idxs.astype(jnp.int32)
    table_lin = _table_linear(emb_table)
    idx_g = ((idx & ~1023) | ((idx & 511) << 1) | ((idx >> 9) & 1))
    idx_pad_flat = jnp.pad(idx_g, ((0, 0), (0, LPAD - L))).reshape(-1)
    sums = _sc_gather_sum(table_lin, idx_pad_flat)
    return _tc_combine(idx, sums, W.T, b.reshape(1, OUT_DIM),
                       emb_table[1:2])


# speedup vs baseline: 1.1248x; 1.1248x over previous
import math

import jax
import jax.numpy as jnp
import numpy as np
from jax import lax
from jax.experimental import pallas as pl
from jax.experimental.pallas import tpu as pltpu
from jax.experimental.pallas import tpu_sc as plsc

VOCAB = 100000
EMBED = 64
OUT_DIM = 128
B = 4096
L = 200
MAX_LEN = 200

NUM_CORES = 2
NUM_SUBCORES = 16
NW = NUM_CORES * NUM_SUBCORES
B_PER_W = B // NW
G0, G1 = 128, 72
LPAD = 256
CHUNK = B_PER_W * LPAD


def _positional_table():
    position = np.arange(MAX_LEN, dtype=np.float32)[:, None]
    div_term = np.exp(
        np.arange(0, EMBED, 2, dtype=np.float32) * (-math.log(10000.0) / EMBED))
    pe = np.zeros((MAX_LEN, EMBED), dtype=np.float32)
    pe[:, 0::2] = np.sin(position * div_term)
    pe[:, 1::2] = np.cos(position * div_term)
    return pe


_PE = _positional_table()


def _sc_gather_sum(emb_table, idx_pad_flat):
    mesh = plsc.VectorSubcoreMesh(core_axis_name="c", subcore_axis_name="s")

    def body(table_hbm, idx_hbm, out_hbm, idx_v, rows_v, out_v, sem0, sem1):
        wid = lax.axis_index("s") * NUM_CORES + lax.axis_index("c")
        pltpu.sync_copy(idx_hbm.at[pl.ds(wid * CHUNK, CHUNK)], idx_v)

        def gather(slot, r, sem):
            base = r * LPAD
            pltpu.make_async_copy(
                table_hbm.at[idx_v.at[pl.ds(base, G0)]],
                rows_v.at[slot, pl.ds(0, G0)], sem).start()
            pltpu.make_async_copy(
                table_hbm.at[idx_v.at[pl.ds(base + G0, G1)]],
                rows_v.at[slot, pl.ds(G0, G1)], sem).start()

        def drain(slot, sem):
            pltpu.make_async_copy(
                table_hbm.at[idx_v.at[pl.ds(0, G0)]],
                rows_v.at[slot, pl.ds(0, G0)], sem).wait()
            pltpu.make_async_copy(
                table_hbm.at[idx_v.at[pl.ds(0, G1)]],
                rows_v.at[slot, pl.ds(G0, G1)], sem).wait()

        def reduce_into(slot, r):
            def red(l, accs):
                b0, b1, b2, b3, c0, c1, c2, c3 = accs
                row = 4 * l
                b0 = b0 + rows_v[slot, row, pl.ds(0, 16)]
                b1 = b1 + rows_v[slot, row, pl.ds(16, 16)]
                b2 = b2 + rows_v[slot, row, pl.ds(32, 16)]
                b3 = b3 + rows_v[slot, row, pl.ds(48, 16)]
                c0 = c0 + rows_v[slot, row + 1, pl.ds(0, 16)]
                c1 = c1 + rows_v[slot, row + 1, pl.ds(16, 16)]
                c2 = c2 + rows_v[slot, row + 1, pl.ds(32, 16)]
                c3 = c3 + rows_v[slot, row + 1, pl.ds(48, 16)]
                b0 = b0 + rows_v[slot, row + 2, pl.ds(0, 16)]
                b1 = b1 + rows_v[slot, row + 2, pl.ds(16, 16)]
                b2 = b2 + rows_v[slot, row + 2, pl.ds(32, 16)]
                b3 = b3 + rows_v[slot, row + 2, pl.ds(48, 16)]
                c0 = c0 + rows_v[slot, row + 3, pl.ds(0, 16)]
                c1 = c1 + rows_v[slot, row + 3, pl.ds(16, 16)]
                c2 = c2 + rows_v[slot, row + 3, pl.ds(32, 16)]
                c3 = c3 + rows_v[slot, row + 3, pl.ds(48, 16)]
                return (b0, b1, b2, b3, c0, c1, c2, c3)

            z = jnp.zeros((16,), jnp.float32)
            accs = lax.fori_loop(0, L // 4, red, (z,) * 8)
            out_v[r, pl.ds(0, 16)] = accs[0] + accs[4]
            out_v[r, pl.ds(16, 16)] = accs[1] + accs[5]
            out_v[r, pl.ds(32, 16)] = accs[2] + accs[6]
            out_v[r, pl.ds(48, 16)] = accs[3] + accs[7]

        gather(0, 0, sem0)

        def pair_body(i, carry):
            r0 = 2 * i
            gather(1, r0 + 1, sem1)
            drain(0, sem0)
            reduce_into(0, r0)

            @pl.when(r0 + 2 < B_PER_W)
            def _():
                gather(0, r0 + 2, sem0)

            drain(1, sem1)
            reduce_into(1, r0 + 1)
            return carry

        lax.fori_loop(0, B_PER_W // 2, pair_body, 0)
        pltpu.sync_copy(out_v, out_hbm.at[pl.ds(wid * B_PER_W, B_PER_W)])

    call = pl.kernel(
        body,
        out_type=jax.ShapeDtypeStruct((B, EMBED), jnp.float32),
        mesh=mesh,
        compiler_params=pltpu.CompilerParams(use_tc_tiling_on_sc=False),
        scratch_types=[
            pltpu.VMEM((CHUNK,), jnp.int32),
            pltpu.VMEM((2, L, EMBED), jnp.float32),
            pltpu.VMEM((B_PER_W, EMBED), jnp.float32),
            pltpu.SemaphoreType.DMA,
            pltpu.SemaphoreType.DMA,
        ],
    )
    return call(emb_table, idx_pad_flat)


TROWS = 2048
TGRID = -(-VOCAB // TROWS)


VOCAB_PAD = TGRID * TROWS


def _table_lin_body(tt_ref, out_ref):
    t = tt_ref[...]
    a = t[:, : TROWS // 2].T
    b = t[:, TROWS // 2 :].T
    out_ref[...] = jnp.concatenate([a, b], axis=1)


def _table_linear(emb_table):
    tt = emb_table.T
    out = pl.pallas_call(
        _table_lin_body,
        grid=(TGRID,),
        in_specs=[pl.BlockSpec((EMBED, TROWS), lambda i: (0, i))],
        out_specs=pl.BlockSpec((TROWS // 2, 2 * EMBED), lambda i: (i, 0)),
        out_shape=jax.ShapeDtypeStruct((VOCAB_PAD // 2, 2 * EMBED), jnp.float32),
    )(tt)
    return out.reshape(VOCAB_PAD, EMBED)


def _tc_combine_body(idx_ref, sums_ref, pe_ref, wt_ref, b_ref, e1_ref, out_ref):
    idx = idx_ref[...]
    m = jnp.logical_and(idx != 0, idx != 1).astype(jnp.float32)
    n_eos = jnp.sum((idx == 1).astype(jnp.float32), axis=1, keepdims=True)
    count = jnp.sum(m, axis=1, keepdims=True)
    pe_sum = jnp.dot(m, pe_ref[...], preferred_element_type=jnp.float32)
    s_masked = sums_ref[...] - n_eos * e1_ref[...]
    pooled = (8.0 * s_masked + pe_sum) / (count + 1e-6)
    out_ref[...] = (
        jnp.dot(pooled, wt_ref[...], preferred_element_type=jnp.float32)
        + b_ref[...])


def _tc_combine(idx, sums, wt, bvec, e1):
    return pl.pallas_call(
        _tc_combine_body,
        out_shape=jax.ShapeDtypeStruct((B, OUT_DIM), jnp.float32),
    )(idx, sums, jnp.asarray(_PE), wt, bvec, e1)


def kernel(src_tok_idxs, emb_table, W, b):
    idx = src_tok_idxs.astype(jnp.int32)
    table_lin = _table_linear(emb_table)
    idx_g = ((idx & ~(TROWS - 1)) | ((idx & (TROWS // 2 - 1)) << 1)
             | ((idx >> 10) & 1))
    idx_pad_flat = jnp.pad(idx_g, ((0, 0), (0, LPAD - L))).reshape(-1)
    sums = _sc_gather_sum(table_lin, idx_pad_flat)
    return _tc_combine(idx, sums, W.T, b.reshape(1, OUT_DIM),
                       emb_table[1:2])


# speedup vs baseline: 1.1973x; 1.0645x over previous
import math

import jax
import jax.numpy as jnp
import numpy as np
from jax import lax
from jax.experimental import pallas as pl
from jax.experimental.pallas import tpu as pltpu
from jax.experimental.pallas import tpu_sc as plsc

VOCAB = 100000
EMBED = 64
OUT_DIM = 128
B = 4096
L = 200
MAX_LEN = 200

NUM_CORES = 2
NUM_SUBCORES = 16
NW = NUM_CORES * NUM_SUBCORES
B_PER_W = B // NW
G0, G1 = 128, 72
LPAD = 256
CHUNK = B_PER_W * LPAD


def _positional_table():
    position = np.arange(MAX_LEN, dtype=np.float32)[:, None]
    div_term = np.exp(
        np.arange(0, EMBED, 2, dtype=np.float32) * (-math.log(10000.0) / EMBED))
    pe = np.zeros((MAX_LEN, EMBED), dtype=np.float32)
    pe[:, 0::2] = np.sin(position * div_term)
    pe[:, 1::2] = np.cos(position * div_term)
    return pe


_PE = _positional_table()


def _sc_gather_sum(emb_table, idx_pad_flat):
    mesh = plsc.VectorSubcoreMesh(core_axis_name="c", subcore_axis_name="s")

    def body(table_hbm, idx_hbm, out_hbm, idx_v, rows_v, out_v, sem0, sem1):
        wid = lax.axis_index("s") * NUM_CORES + lax.axis_index("c")
        pltpu.sync_copy(idx_hbm.at[pl.ds(wid * CHUNK, CHUNK)], idx_v)

        def gather(slot, r, sem):
            base = r * LPAD
            pltpu.make_async_copy(
                table_hbm.at[idx_v.at[pl.ds(base, G0)]],
                rows_v.at[slot, pl.ds(0, G0)], sem).start()
            pltpu.make_async_copy(
                table_hbm.at[idx_v.at[pl.ds(base + G0, G1)]],
                rows_v.at[slot, pl.ds(G0, G1)], sem).start()

        def drain(slot, sem):
            pltpu.make_async_copy(
                table_hbm.at[idx_v.at[pl.ds(0, G0)]],
                rows_v.at[slot, pl.ds(0, G0)], sem).wait()
            pltpu.make_async_copy(
                table_hbm.at[idx_v.at[pl.ds(0, G1)]],
                rows_v.at[slot, pl.ds(G0, G1)], sem).wait()

        def reduce_into(slot, r):
            def red(l, accs):
                b0, b1, b2, b3, c0, c1, c2, c3 = accs
                row = 4 * l
                b0 = b0 + rows_v[slot, row, pl.ds(0, 16)]
                b1 = b1 + rows_v[slot, row, pl.ds(16, 16)]
                b2 = b2 + rows_v[slot, row, pl.ds(32, 16)]
                b3 = b3 + rows_v[slot, row, pl.ds(48, 16)]
                c0 = c0 + rows_v[slot, row + 1, pl.ds(0, 16)]
                c1 = c1 + rows_v[slot, row + 1, pl.ds(16, 16)]
                c2 = c2 + rows_v[slot, row + 1, pl.ds(32, 16)]
                c3 = c3 + rows_v[slot, row + 1, pl.ds(48, 16)]
                b0 = b0 + rows_v[slot, row + 2, pl.ds(0, 16)]
                b1 = b1 + rows_v[slot, row + 2, pl.ds(16, 16)]
                b2 = b2 + rows_v[slot, row + 2, pl.ds(32, 16)]
                b3 = b3 + rows_v[slot, row + 2, pl.ds(48, 16)]
                c0 = c0 + rows_v[slot, row + 3, pl.ds(0, 16)]
                c1 = c1 + rows_v[slot, row + 3, pl.ds(16, 16)]
                c2 = c2 + rows_v[slot, row + 3, pl.ds(32, 16)]
                c3 = c3 + rows_v[slot, row + 3, pl.ds(48, 16)]
                return (b0, b1, b2, b3, c0, c1, c2, c3)

            z = jnp.zeros((16,), jnp.float32)
            accs = lax.fori_loop(0, L // 4, red, (z,) * 8)
            out_v[r, pl.ds(0, 16)] = accs[0] + accs[4]
            out_v[r, pl.ds(16, 16)] = accs[1] + accs[5]
            out_v[r, pl.ds(32, 16)] = accs[2] + accs[6]
            out_v[r, pl.ds(48, 16)] = accs[3] + accs[7]

        gather(0, 0, sem0)

        def pair_body(i, carry):
            r0 = 2 * i
            gather(1, r0 + 1, sem1)
            drain(0, sem0)
            reduce_into(0, r0)

            @pl.when(r0 + 2 < B_PER_W)
            def _():
                gather(0, r0 + 2, sem0)

            drain(1, sem1)
            reduce_into(1, r0 + 1)
            return carry

        lax.fori_loop(0, B_PER_W // 2, pair_body, 0)
        pltpu.sync_copy(out_v, out_hbm.at[pl.ds(wid * B_PER_W, B_PER_W)])

    call = pl.kernel(
        body,
        out_type=jax.ShapeDtypeStruct((B, EMBED), jnp.float32),
        mesh=mesh,
        compiler_params=pltpu.CompilerParams(use_tc_tiling_on_sc=False),
        scratch_types=[
            pltpu.VMEM((CHUNK,), jnp.int32),
            pltpu.VMEM((2, L, EMBED), jnp.float32),
            pltpu.VMEM((B_PER_W, EMBED), jnp.float32),
            pltpu.SemaphoreType.DMA,
            pltpu.SemaphoreType.DMA,
        ],
    )
    return call(emb_table, idx_pad_flat)


TROWS = 4096
TGRID = -(-VOCAB // TROWS)


VOCAB_PAD = TGRID * TROWS


def _table_lin_body(tt_ref, out_ref):
    t = tt_ref[...]
    a = t[:, : TROWS // 2].T
    b = t[:, TROWS // 2 :].T
    out_ref[...] = jnp.concatenate([a, b], axis=1)


def _table_linear(emb_table):
    tt = emb_table.T
    out = pl.pallas_call(
        _table_lin_body,
        grid=(TGRID,),
        in_specs=[pl.BlockSpec((EMBED, TROWS), lambda i: (0, i))],
        out_specs=pl.BlockSpec((TROWS // 2, 2 * EMBED), lambda i: (i, 0)),
        out_shape=jax.ShapeDtypeStruct((VOCAB_PAD // 2, 2 * EMBED), jnp.float32),
    )(tt)
    return out.reshape(VOCAB_PAD, EMBED)


def _tc_combine_body(idx_ref, sums_ref, pe_ref, wt_ref, b_ref, e1_ref, out_ref):
    idx = idx_ref[...]
    m = jnp.logical_and(idx != 0, idx != 1).astype(jnp.float32)
    n_eos = jnp.sum((idx == 1).astype(jnp.float32), axis=1, keepdims=True)
    count = jnp.sum(m, axis=1, keepdims=True)
    pe_sum = jnp.dot(m, pe_ref[...], preferred_element_type=jnp.float32)
    s_masked = sums_ref[...] - n_eos * e1_ref[...]
    pooled = (8.0 * s_masked + pe_sum) / (count + 1e-6)
    out_ref[...] = (
        jnp.dot(pooled, wt_ref[...], preferred_element_type=jnp.float32)
        + b_ref[...])


def _tc_combine(idx, sums, wt, bvec, e1):
    return pl.pallas_call(
        _tc_combine_body,
        out_shape=jax.ShapeDtypeStruct((B, OUT_DIM), jnp.float32),
    )(idx, sums, jnp.asarray(_PE), wt, bvec, e1)


def kernel(src_tok_idxs, emb_table, W, b):
    idx = src_tok_idxs.astype(jnp.int32)
    table_lin = _table_linear(emb_table)
    idx_g = ((idx & ~(TROWS - 1)) | ((idx & (TROWS // 2 - 1)) << 1)
             | ((idx >> 11) & 1))
    idx_pad_flat = jnp.pad(idx_g, ((0, 0), (0, LPAD - L))).reshape(-1)
    sums = _sc_gather_sum(table_lin, idx_pad_flat)
    return _tc_combine(idx, sums, W.T, b.reshape(1, OUT_DIM),
                       emb_table[1:2])


# speedup vs baseline: 1.2343x; 1.0309x over previous
import math

import jax
import jax.numpy as jnp
import numpy as np
from jax import lax
from jax.experimental import pallas as pl
from jax.experimental.pallas import tpu as pltpu
from jax.experimental.pallas import tpu_sc as plsc

VOCAB = 100000
EMBED = 64
OUT_DIM = 128
B = 4096
L = 200
MAX_LEN = 200

NUM_CORES = 2
NUM_SUBCORES = 16
NW = NUM_CORES * NUM_SUBCORES
B_PER_W = B // NW
G0, G1 = 128, 72
LPAD = 256
CHUNK = B_PER_W * LPAD


def _positional_table():
    position = np.arange(MAX_LEN, dtype=np.float32)[:, None]
    div_term = np.exp(
        np.arange(0, EMBED, 2, dtype=np.float32) * (-math.log(10000.0) / EMBED))
    pe = np.zeros((MAX_LEN, EMBED), dtype=np.float32)
    pe[:, 0::2] = np.sin(position * div_term)
    pe[:, 1::2] = np.cos(position * div_term)
    return pe


_PE = _positional_table()


def _sc_gather_sum(emb_table, idx_pad_flat):
    mesh = plsc.VectorSubcoreMesh(core_axis_name="c", subcore_axis_name="s")

    def body(table_hbm, idx_hbm, out_hbm, idx_v, rows_v, out_v, sem0, sem1):
        wid = lax.axis_index("s") * NUM_CORES + lax.axis_index("c")
        pltpu.sync_copy(idx_hbm.at[pl.ds(wid * CHUNK, CHUNK)], idx_v)

        def gather(slot, r, sem):
            base = r * LPAD
            pltpu.make_async_copy(
                table_hbm.at[idx_v.at[pl.ds(base, G0)]],
                rows_v.at[slot, pl.ds(0, G0)], sem).start()
            pltpu.make_async_copy(
                table_hbm.at[idx_v.at[pl.ds(base + G0, G1)]],
                rows_v.at[slot, pl.ds(G0, G1)], sem).start()

        def drain(slot, sem):
            pltpu.make_async_copy(
                table_hbm.at[idx_v.at[pl.ds(0, G0)]],
                rows_v.at[slot, pl.ds(0, G0)], sem).wait()
            pltpu.make_async_copy(
                table_hbm.at[idx_v.at[pl.ds(0, G1)]],
                rows_v.at[slot, pl.ds(G0, G1)], sem).wait()

        def reduce_into(slot, r):
            def red(l, accs):
                b0, b1, b2, b3, c0, c1, c2, c3 = accs
                row = 4 * l
                b0 = b0 + rows_v[slot, row, pl.ds(0, 16)]
                b1 = b1 + rows_v[slot, row, pl.ds(16, 16)]
                b2 = b2 + rows_v[slot, row, pl.ds(32, 16)]
                b3 = b3 + rows_v[slot, row, pl.ds(48, 16)]
                c0 = c0 + rows_v[slot, row + 1, pl.ds(0, 16)]
                c1 = c1 + rows_v[slot, row + 1, pl.ds(16, 16)]
                c2 = c2 + rows_v[slot, row + 1, pl.ds(32, 16)]
                c3 = c3 + rows_v[slot, row + 1, pl.ds(48, 16)]
                b0 = b0 + rows_v[slot, row + 2, pl.ds(0, 16)]
                b1 = b1 + rows_v[slot, row + 2, pl.ds(16, 16)]
                b2 = b2 + rows_v[slot, row + 2, pl.ds(32, 16)]
                b3 = b3 + rows_v[slot, row + 2, pl.ds(48, 16)]
                c0 = c0 + rows_v[slot, row + 3, pl.ds(0, 16)]
                c1 = c1 + rows_v[slot, row + 3, pl.ds(16, 16)]
                c2 = c2 + rows_v[slot, row + 3, pl.ds(32, 16)]
                c3 = c3 + rows_v[slot, row + 3, pl.ds(48, 16)]
                return (b0, b1, b2, b3, c0, c1, c2, c3)

            z = jnp.zeros((16,), jnp.float32)
            accs = lax.fori_loop(0, L // 4, red, (z,) * 8)
            out_v[r, pl.ds(0, 16)] = accs[0] + accs[4]
            out_v[r, pl.ds(16, 16)] = accs[1] + accs[5]
            out_v[r, pl.ds(32, 16)] = accs[2] + accs[6]
            out_v[r, pl.ds(48, 16)] = accs[3] + accs[7]

        gather(0, 0, sem0)

        def pair_body(i, carry):
            r0 = 2 * i
            gather(1, r0 + 1, sem1)
            drain(0, sem0)
            reduce_into(0, r0)

            @pl.when(r0 + 2 < B_PER_W)
            def _():
                gather(0, r0 + 2, sem0)

            drain(1, sem1)
            reduce_into(1, r0 + 1)
            return carry

        lax.fori_loop(0, B_PER_W // 2, pair_body, 0)
        pltpu.sync_copy(out_v, out_hbm.at[pl.ds(wid * B_PER_W, B_PER_W)])

    call = pl.kernel(
        body,
        out_type=jax.ShapeDtypeStruct((B, EMBED), jnp.float32),
        mesh=mesh,
        compiler_params=pltpu.CompilerParams(use_tc_tiling_on_sc=False),
        scratch_types=[
            pltpu.VMEM((CHUNK,), jnp.int32),
            pltpu.VMEM((2, L, EMBED), jnp.float32),
            pltpu.VMEM((B_PER_W, EMBED), jnp.float32),
            pltpu.SemaphoreType.DMA,
            pltpu.SemaphoreType.DMA,
        ],
    )
    return call(emb_table, idx_pad_flat)


TROWS = 8192
TGRID = -(-VOCAB // TROWS)


VOCAB_PAD = TGRID * TROWS


def _table_lin_body(tt_ref, out_ref):
    t = tt_ref[...]
    a = t[:, : TROWS // 2].T
    b = t[:, TROWS // 2 :].T
    out_ref[...] = jnp.concatenate([a, b], axis=1)


def _table_linear(emb_table):
    tt = emb_table.T
    out = pl.pallas_call(
        _table_lin_body,
        grid=(TGRID,),
        in_specs=[pl.BlockSpec((EMBED, TROWS), lambda i: (0, i))],
        out_specs=pl.BlockSpec((TROWS // 2, 2 * EMBED), lambda i: (i, 0)),
        out_shape=jax.ShapeDtypeStruct((VOCAB_PAD // 2, 2 * EMBED), jnp.float32),
    )(tt)
    return out.reshape(VOCAB_PAD, EMBED)


def _tc_combine_body(idx_ref, sums_ref, pe_ref, wt_ref, b_ref, e1_ref, out_ref):
    idx = idx_ref[...]
    m = jnp.logical_and(idx != 0, idx != 1).astype(jnp.float32)
    n_eos = jnp.sum((idx == 1).astype(jnp.float32), axis=1, keepdims=True)
    count = jnp.sum(m, axis=1, keepdims=True)
    pe_sum = jnp.dot(m, pe_ref[...], preferred_element_type=jnp.float32)
    s_masked = sums_ref[...] - n_eos * e1_ref[...]
    pooled = (8.0 * s_masked + pe_sum) / (count + 1e-6)
    out_ref[...] = (
        jnp.dot(pooled, wt_ref[...], preferred_element_type=jnp.float32)
        + b_ref[...])


def _tc_combine(idx, sums, wt, bvec, e1):
    return pl.pallas_call(
        _tc_combine_body,
        out_shape=jax.ShapeDtypeStruct((B, OUT_DIM), jnp.float32),
    )(idx, sums, jnp.asarray(_PE), wt, bvec, e1)


def kernel(src_tok_idxs, emb_table, W, b):
    idx = src_tok_idxs.astype(jnp.int32)
    table_lin = _table_linear(emb_table)
    idx_g = ((idx & ~(TROWS - 1)) | ((idx & (TROWS // 2 - 1)) << 1)
             | ((idx >> 12) & 1))
    idx_pad_flat = jnp.pad(idx_g, ((0, 0), (0, LPAD - L))).reshape(-1)
    sums = _sc_gather_sum(table_lin, idx_pad_flat)
    return _tc_combine(idx, sums, W.T, b.reshape(1, OUT_DIM),
                       emb_table[1:2])


# speedup vs baseline: 1.2394x; 1.0042x over previous
import math

import jax
import jax.numpy as jnp
import numpy as np
from jax import lax
from jax.experimental import pallas as pl
from jax.experimental.pallas import tpu as pltpu
from jax.experimental.pallas import tpu_sc as plsc

VOCAB = 100000
EMBED = 64
OUT_DIM = 128
B = 4096
L = 200
MAX_LEN = 200

NUM_CORES = 2
NUM_SUBCORES = 16
NW = NUM_CORES * NUM_SUBCORES
B_PER_W = B // NW
G0, G1 = 128, 72
LPAD = 256
CHUNK = B_PER_W * LPAD


def _positional_table():
    position = np.arange(MAX_LEN, dtype=np.float32)[:, None]
    div_term = np.exp(
        np.arange(0, EMBED, 2, dtype=np.float32) * (-math.log(10000.0) / EMBED))
    pe = np.zeros((MAX_LEN, EMBED), dtype=np.float32)
    pe[:, 0::2] = np.sin(position * div_term)
    pe[:, 1::2] = np.cos(position * div_term)
    return pe


_PE = _positional_table()


def _sc_gather_sum(emb_table, idx_pad_flat):
    mesh = plsc.VectorSubcoreMesh(core_axis_name="c", subcore_axis_name="s")

    def body(table_hbm, idx_hbm, out_hbm, idx_v, rows_v, out_v, sem0, sem1):
        wid = lax.axis_index("s") * NUM_CORES + lax.axis_index("c")
        pltpu.sync_copy(idx_hbm.at[pl.ds(wid * CHUNK, CHUNK)], idx_v)

        def gather(slot, r, sem):
            base = r * LPAD
            pltpu.make_async_copy(
                table_hbm.at[idx_v.at[pl.ds(base, G0)]],
                rows_v.at[slot, pl.ds(0, G0)], sem).start()
            pltpu.make_async_copy(
                table_hbm.at[idx_v.at[pl.ds(base + G0, G1)]],
                rows_v.at[slot, pl.ds(G0, G1)], sem).start()

        def drain(slot, sem):
            pltpu.make_async_copy(
                table_hbm.at[idx_v.at[pl.ds(0, G0)]],
                rows_v.at[slot, pl.ds(0, G0)], sem).wait()
            pltpu.make_async_copy(
                table_hbm.at[idx_v.at[pl.ds(0, G1)]],
                rows_v.at[slot, pl.ds(G0, G1)], sem).wait()

        def reduce_into(slot, r):
            def red(l, accs):
                b0, b1, b2, b3, c0, c1, c2, c3 = accs
                row = 4 * l
                b0 = b0 + rows_v[slot, row, pl.ds(0, 16)]
                b1 = b1 + rows_v[slot, row, pl.ds(16, 16)]
                b2 = b2 + rows_v[slot, row, pl.ds(32, 16)]
                b3 = b3 + rows_v[slot, row, pl.ds(48, 16)]
                c0 = c0 + rows_v[slot, row + 1, pl.ds(0, 16)]
                c1 = c1 + rows_v[slot, row + 1, pl.ds(16, 16)]
                c2 = c2 + rows_v[slot, row + 1, pl.ds(32, 16)]
                c3 = c3 + rows_v[slot, row + 1, pl.ds(48, 16)]
                b0 = b0 + rows_v[slot, row + 2, pl.ds(0, 16)]
                b1 = b1 + rows_v[slot, row + 2, pl.ds(16, 16)]
                b2 = b2 + rows_v[slot, row + 2, pl.ds(32, 16)]
                b3 = b3 + rows_v[slot, row + 2, pl.ds(48, 16)]
                c0 = c0 + rows_v[slot, row + 3, pl.ds(0, 16)]
                c1 = c1 + rows_v[slot, row + 3, pl.ds(16, 16)]
                c2 = c2 + rows_v[slot, row + 3, pl.ds(32, 16)]
                c3 = c3 + rows_v[slot, row + 3, pl.ds(48, 16)]
                return (b0, b1, b2, b3, c0, c1, c2, c3)

            z = jnp.zeros((16,), jnp.float32)
            accs = lax.fori_loop(0, L // 4, red, (z,) * 8)
            out_v[r, pl.ds(0, 16)] = accs[0] + accs[4]
            out_v[r, pl.ds(16, 16)] = accs[1] + accs[5]
            out_v[r, pl.ds(32, 16)] = accs[2] + accs[6]
            out_v[r, pl.ds(48, 16)] = accs[3] + accs[7]

        gather(0, 0, sem0)

        def pair_body(i, carry):
            r0 = 2 * i
            gather(1, r0 + 1, sem1)
            drain(0, sem0)
            reduce_into(0, r0)

            @pl.when(r0 + 2 < B_PER_W)
            def _():
                gather(0, r0 + 2, sem0)

            drain(1, sem1)
            reduce_into(1, r0 + 1)
            return carry

        lax.fori_loop(0, B_PER_W // 2, pair_body, 0)
        pltpu.sync_copy(out_v, out_hbm.at[pl.ds(wid * B_PER_W, B_PER_W)])

    call = pl.kernel(
        body,
        out_type=jax.ShapeDtypeStruct((B, EMBED), jnp.float32),
        mesh=mesh,
        compiler_params=pltpu.CompilerParams(use_tc_tiling_on_sc=False),
        scratch_types=[
            pltpu.VMEM((CHUNK,), jnp.int32),
            pltpu.VMEM((2, L, EMBED), jnp.float32),
            pltpu.VMEM((B_PER_W, EMBED), jnp.float32),
            pltpu.SemaphoreType.DMA,
            pltpu.SemaphoreType.DMA,
        ],
    )
    return call(emb_table, idx_pad_flat)


TROWS = 16384
TGRID = -(-VOCAB // TROWS)


VOCAB_PAD = TGRID * TROWS


def _table_lin_body(tt_ref, out_ref):
    t = tt_ref[...]
    a = t[:, : TROWS // 2].T
    b = t[:, TROWS // 2 :].T
    out_ref[...] = jnp.concatenate([a, b], axis=1)


def _table_linear(emb_table):
    tt = emb_table.T
    out = pl.pallas_call(
        _table_lin_body,
        grid=(TGRID,),
        in_specs=[pl.BlockSpec((EMBED, TROWS), lambda i: (0, i))],
        out_specs=pl.BlockSpec((TROWS // 2, 2 * EMBED), lambda i: (i, 0)),
        out_shape=jax.ShapeDtypeStruct((VOCAB_PAD // 2, 2 * EMBED), jnp.float32),
    )(tt)
    return out.reshape(VOCAB_PAD, EMBED)


def _tc_combine_body(idx_ref, sums_ref, pe_ref, wt_ref, b_ref, e1_ref, out_ref):
    idx = idx_ref[...]
    m = jnp.logical_and(idx != 0, idx != 1).astype(jnp.float32)
    n_eos = jnp.sum((idx == 1).astype(jnp.float32), axis=1, keepdims=True)
    count = jnp.sum(m, axis=1, keepdims=True)
    pe_sum = jnp.dot(m, pe_ref[...], preferred_element_type=jnp.float32)
    s_masked = sums_ref[...] - n_eos * e1_ref[...]
    pooled = (8.0 * s_masked + pe_sum) / (count + 1e-6)
    out_ref[...] = (
        jnp.dot(pooled, wt_ref[...], preferred_element_type=jnp.float32)
        + b_ref[...])


def _tc_combine(idx, sums, wt, bvec, e1):
    return pl.pallas_call(
        _tc_combine_body,
        out_shape=jax.ShapeDtypeStruct((B, OUT_DIM), jnp.float32),
    )(idx, sums, jnp.asarray(_PE), wt, bvec, e1)


def kernel(src_tok_idxs, emb_table, W, b):
    idx = src_tok_idxs.astype(jnp.int32)
    table_lin = _table_linear(emb_table)
    idx_g = ((idx & ~(TROWS - 1)) | ((idx & (TROWS // 2 - 1)) << 1)
             | ((idx >> 13) & 1))
    idx_pad_flat = jnp.pad(idx_g, ((0, 0), (0, LPAD - L))).reshape(-1)
    sums = _sc_gather_sum(table_lin, idx_pad_flat)
    return _tc_combine(idx, sums, W.T, b.reshape(1, OUT_DIM),
                       emb_table[1:2])
